# baseline (speedup 1.0000x reference)
import functools

import jax
import jax.numpy as jnp
from jax import lax
from jax.experimental import pallas as pl
from jax.experimental.pallas import tpu as pltpu
from jax.experimental.pallas import tpu_sc as plsc

N = 10000
NP = 10240
E = 320000
F = 128
H = 32
C = 8

NC = 2
NS = 16
L = 16
CHUNK = 128
CPT = 80
EP = NC * NS * CHUNK * CPT
ER = EP // CHUNK
RPT = NP // NS

R = 2048
GRID = NP // R

_sc_mesh = plsc.VectorSubcoreMesh(core_axis_name="c", subcore_axis_name="s")
_sc_params = pltpu.CompilerParams(needs_layout_passes=False,
                                  use_tc_tiling_on_sc=False)


DR = ER // NS


@functools.partial(
    pl.kernel,
    out_type=(jax.ShapeDtypeStruct((NC, NP, H), jnp.float32),
              jax.ShapeDtypeStruct((NC, NP), jnp.float32)),
    mesh=_sc_mesh,
    scratch_types=[
        pltpu.VMEM((DR * CHUNK,), jnp.int32),
        pltpu.VMEM((DR * CHUNK,), jnp.float32),
        pltpu.VMEM((NP,), jnp.float32),
        pltpu.VMEM((NP,), jnp.float32),
        pltpu.VMEM((RPT,), jnp.float32),
        pltpu.VMEM((CHUNK,), jnp.int32),
        pltpu.VMEM((CHUNK,), jnp.int32),
        pltpu.VMEM((CHUNK,), jnp.int32),
        pltpu.VMEM((CHUNK,), jnp.int32),
        pltpu.VMEM((CHUNK,), jnp.float32),
        pltpu.VMEM((CHUNK,), jnp.float32),
        pltpu.VMEM((CHUNK, H), jnp.float32),
        pltpu.VMEM((CHUNK, H), jnp.float32),
        pltpu.VMEM((CHUNK, H), jnp.float32),
        pltpu.VMEM((CHUNK, H), jnp.float32),
        pltpu.VMEM_SHARED((NS, NP), jnp.float32),
        pltpu.VMEM_SHARED((NP, H), jnp.float32),
        pltpu.SemaphoreType.DMA,
        pltpu.SemaphoreType.DMA,
        pltpu.SemaphoreType.DMA,
        pltpu.SemaphoreType.DMA,
    ],
    compiler_params=_sc_params,
)
def _sc_deg_scatter(h_hbm, src_hbm, dst_hbm, ew_hbm,
                    out_hbm, dis_hbm,
                    gidx, gval, dacc, dtbl, red,
                    sidx0, sidx1, didx0, didx1, ewv0, ewv1,
                    rg0, rg1, rs0, rs1, shared, acc,
                    gsem0, gsem1, ssem0, ssem1):
    c = lax.axis_index("c")
    s = lax.axis_index("s")
    zero = jnp.zeros((L,), jnp.float32)
    ebase = (c * NS + s) * CPT * CHUNK

    pltpu.sync_copy(src_hbm.at[pl.ds(ebase, CHUNK)], sidx0)
    pltpu.async_copy(h_hbm.at[sidx0], rg0, gsem0)
    pltpu.sync_copy(src_hbm.at[pl.ds(ebase + CHUNK, CHUNK)], sidx1)
    pltpu.async_copy(h_hbm.at[sidx1], rg1, gsem1)

    def zdacc(i, carry):
        dacc[pl.ds(i * L, L)] = zero
        return carry

    lax.fori_loop(0, NP // L, zdacc, 0)
    pltpu.sync_copy(dst_hbm.at[pl.ds(s * DR * CHUNK, DR * CHUNK)], gidx)
    pltpu.sync_copy(ew_hbm.at[pl.ds(s * DR * CHUNK, DR * CHUNK)], gval)

    def dchunk(m, carry):
        for k in range(8):
            off = m * (8 * L) + k * L
            dvec = gidx[pl.ds(off, L)]
            wvec = gval[pl.ds(off, L)]
            plsc.addupdate_scatter(dacc, [dvec], wvec)
        return carry

    lax.fori_loop(0, DR * CHUNK // (8 * L), dchunk, 0)

    def zrow(e, carry):
        rs0[e, pl.ds(0, L)] = zero
        rs0[e, pl.ds(L, L)] = zero
        return carry

    lax.fori_loop(0, CHUNK, zrow, 0)

    def zacc(i, carry):
        pltpu.sync_copy(rs0, acc.at[pl.ds(s * RPT + i * CHUNK, CHUNK)])
        return carry

    lax.fori_loop(0, RPT // CHUNK, zacc, 0)

    pltpu.sync_copy(dacc, shared.at[s])
    plsc.subcore_barrier()

    def rinit(i, carry):
        red[pl.ds(i * L, L)] = zero
        return carry

    lax.fori_loop(0, RPT // L, rinit, 0)

    def raccum(t, carry):
        pltpu.sync_copy(shared.at[t, pl.ds(s * RPT, RPT)],
                        dacc.at[pl.ds(0, RPT)])

        def radd(i, icarry):
            red[pl.ds(i * L, L)] = (red[pl.ds(i * L, L)]
                                    + dacc[pl.ds(i * L, L)])
            return icarry

        lax.fori_loop(0, RPT // L, radd, 0)
        return carry

    lax.fori_loop(0, NS, raccum, 0)

    magic = jnp.full((L,), 0x5F3759DF, jnp.int32)

    def dis_slice(i, carry):
        d = red[pl.ds(i * L, L)] + 1.0
        iv = plsc.bitcast(d, jnp.int32)
        y = plsc.bitcast(magic - lax.shift_right_arithmetic(iv, 1),
                         jnp.float32)
        for _ in range(3):
            y = y * (1.5 - 0.5 * d * y * y)
        red[pl.ds(i * L, L)] = y
        return carry

    lax.fori_loop(0, RPT // L, dis_slice, 0)
    pltpu.sync_copy(red, dis_hbm.at[c, pl.ds(s * RPT, RPT)])
    pltpu.sync_copy(red, shared.at[0, pl.ds(s * RPT, RPT)])
    plsc.subcore_barrier()
    pltpu.sync_copy(shared.at[0], dtbl)

    sidxs = (sidx0, sidx1)
    didxs = (didx0, didx1)
    ewvs = (ewv0, ewv1)
    rgs = (rg0, rg1)
    rss = (rs0, rs1)
    gsems = (gsem0, gsem1)
    ssems = (ssem0, ssem1)

    def emit_pair(jj, with_sdrain, issue_next):
        for b in (0, 1):
            j = jj * 2 + b
            base = ebase + j * CHUNK
            sidxb, didxb, ewvb = sidxs[b], didxs[b], ewvs[b]
            rg, rs, gsem, ssem = rgs[b], rss[b], gsems[b], ssems[b]
            pltpu.make_async_copy(h_hbm.at[sidxb], rg, gsem).wait()
            if with_sdrain:
                pltpu.make_async_copy(rs, acc.at[didxb], ssem).wait()
            pltpu.sync_copy(ew_hbm.at[pl.ds(base, CHUNK)], ewvb)
            pltpu.sync_copy(dst_hbm.at[pl.ds(base, CHUNK)], didxb)

            for k in range(CHUNK // L):
                svec = sidxb[pl.ds(k * L, L)]
                disv = plsc.load_gather(dtbl, [svec])
                ewvb[pl.ds(k * L, L)] = ewvb[pl.ds(k * L, L)] * disv

            def escale(e, icarry):
                eidx = lax.broadcast(e, (L,))
                wv = plsc.load_gather(ewvb, [eidx])
                rs[e, pl.ds(0, L)] = rg[e, pl.ds(0, L)] * wv
                rs[e, pl.ds(L, L)] = rg[e, pl.ds(L, L)] * wv
                return icarry

            lax.fori_loop(0, CHUNK, escale, 0)
            pltpu.async_copy(rs, acc.at[didxb], ssem, add=True)
            if issue_next:
                base2 = ebase + (j + 2) * CHUNK
                pltpu.sync_copy(src_hbm.at[pl.ds(base2, CHUNK)], sidxb)
                pltpu.async_copy(h_hbm.at[sidxb], rg, gsem)

    emit_pair(0, False, True)

    def pair_body(jj, carry):
        emit_pair(jj, True, True)
        return carry

    lax.fori_loop(1, CPT // 2 - 1, pair_body, 0)
    emit_pair(CPT // 2 - 1, True, False)

    pltpu.make_async_copy(rs0, acc.at[didx0], ssem0).wait()
    pltpu.make_async_copy(rs1, acc.at[didx1], ssem1).wait()
    plsc.subcore_barrier()
    pltpu.sync_copy(acc.at[pl.ds(s * RPT, RPT)],
                    out_hbm.at[c, pl.ds(s * RPT, RPT)])


@functools.partial(
    pl.kernel,
    out_type=jax.ShapeDtypeStruct((NC, NP, H), jnp.float32),
    mesh=_sc_mesh,
    scratch_types=[
        pltpu.VMEM((CHUNK,), jnp.int32),
        pltpu.VMEM((CHUNK,), jnp.int32),
        pltpu.VMEM((CHUNK,), jnp.int32),
        pltpu.VMEM((CHUNK,), jnp.int32),
        pltpu.VMEM((CHUNK,), jnp.float32),
        pltpu.VMEM((CHUNK,), jnp.float32),
        pltpu.VMEM((CHUNK, H), jnp.float32),
        pltpu.VMEM((CHUNK, H), jnp.float32),
        pltpu.VMEM((CHUNK, H), jnp.float32),
        pltpu.VMEM((CHUNK, H), jnp.float32),
        pltpu.VMEM_SHARED((NP, H), jnp.float32),
        pltpu.SemaphoreType.DMA,
        pltpu.SemaphoreType.DMA,
        pltpu.SemaphoreType.DMA,
        pltpu.SemaphoreType.DMA,
    ],
    compiler_params=_sc_params,
)
def _sc_scatter(g_hbm, src_hbm, dst_hbm, ew_hbm, out_hbm,
                sidx0, sidx1, didx0, didx1, ewv0, ewv1,
                rg0, rg1, rs0, rs1, acc,
                gsem0, gsem1, ssem0, ssem1):
    c = lax.axis_index("c")
    s = lax.axis_index("s")
    zero = jnp.zeros((L,), jnp.float32)

    def zrow(e, carry):
        rs0[e, pl.ds(0, L)] = zero
        rs0[e, pl.ds(L, L)] = zero
        return carry

    lax.fori_loop(0, CHUNK, zrow, 0)

    def zacc(i, carry):
        pltpu.sync_copy(rs0, acc.at[pl.ds(s * RPT + i * CHUNK, CHUNK)])
        return carry

    ebase = (c * NS + s) * CPT * CHUNK
    sidxs = (sidx0, sidx1)
    didxs = (didx0, didx1)
    ewvs = (ewv0, ewv1)
    rgs = (rg0, rg1)
    rss = (rs0, rs1)
    gsems = (gsem0, gsem1)
    ssems = (ssem0, ssem1)

    pltpu.sync_copy(src_hbm.at[pl.ds(ebase, CHUNK)], sidx0)
    pltpu.async_copy(g_hbm.at[sidx0], rg0, gsem0)
    pltpu.sync_copy(src_hbm.at[pl.ds(ebase + CHUNK, CHUNK)], sidx1)
    pltpu.async_copy(g_hbm.at[sidx1], rg1, gsem1)

    lax.fori_loop(0, RPT // CHUNK, zacc, 0)
    plsc.subcore_barrier()

    def emit_pair(jj, with_sdrain, issue_next):
        for b in (0, 1):
            j = jj * 2 + b
            base = ebase + j * CHUNK
            sidxb, didxb, ewvb = sidxs[b], didxs[b], ewvs[b]
            rg, rs, gsem, ssem = rgs[b], rss[b], gsems[b], ssems[b]
            pltpu.make_async_copy(g_hbm.at[sidxb], rg, gsem).wait()
            if with_sdrain:
                pltpu.make_async_copy(rs, acc.at[didxb], ssem).wait()
            pltpu.sync_copy(ew_hbm.at[pl.ds(base, CHUNK)], ewvb)
            pltpu.sync_copy(dst_hbm.at[pl.ds(base, CHUNK)], didxb)

            def escale(e, icarry):
                eidx = lax.broadcast(e, (L,))
                wv = plsc.load_gather(ewvb, [eidx])
                rs[e, pl.ds(0, L)] = rg[e, pl.ds(0, L)] * wv
                rs[e, pl.ds(L, L)] = rg[e, pl.ds(L, L)] * wv
                return icarry

            lax.fori_loop(0, CHUNK, escale, 0)
            pltpu.async_copy(rs, acc.at[didxb], ssem, add=True)
            if issue_next:
                base2 = ebase + (j + 2) * CHUNK
                pltpu.sync_copy(src_hbm.at[pl.ds(base2, CHUNK)], sidxb)
                pltpu.async_copy(g_hbm.at[sidxb], rg, gsem)

    emit_pair(0, False, True)

    def pair_body(jj, carry):
        emit_pair(jj, True, True)
        return carry

    lax.fori_loop(1, CPT // 2 - 1, pair_body, 0)
    emit_pair(CPT // 2 - 1, True, False)

    pltpu.make_async_copy(rs0, acc.at[didx0], ssem0).wait()
    pltpu.make_async_copy(rs1, acc.at[didx1], ssem1).wait()
    plsc.subcore_barrier()
    pltpu.sync_copy(acc.at[pl.ds(s * RPT, RPT)],
                    out_hbm.at[c, pl.ds(s * RPT, RPT)])


def _tc_h1_body(x, w, h):
    h[...] = jnp.dot(x[...], w[...], preferred_element_type=jnp.float32)


_tc_h1 = pl.pallas_call(
    _tc_h1_body,
    grid=(GRID,),
    in_specs=[pl.BlockSpec((R, F), lambda i: (i, 0)),
              pl.BlockSpec((F, H), lambda i: (0, 0))],
    out_specs=pl.BlockSpec((R, H), lambda i: (i, 0)),
    out_shape=jax.ShapeDtypeStruct((NP, H), jnp.float32),
)


def _postact(s0, s1, gpv, dis, b):
    pre = dis * (s0[...] + s1[...] + gpv) + b[...]
    ss = jnp.sum(pre * pre, axis=1, keepdims=True)
    o = pre / jnp.maximum(jnp.sqrt(ss), 1e-12)
    return jnp.maximum(o, 0.0)


_layer_specs = [pl.BlockSpec((R, H), lambda i: (i, 0)),
                pl.BlockSpec((R, H), lambda i: (i, 0)),
                pl.BlockSpec((R, H), lambda i: (i, 0)),
                pl.BlockSpec((R, 1), lambda i: (i, 0)),
                pl.BlockSpec((1, H), lambda i: (0, 0)),
                pl.BlockSpec((H, H), lambda i: (0, 0))]
_layer_outs = [pl.BlockSpec((R, H), lambda i: (i, 0)),
               pl.BlockSpec((R, H), lambda i: (i, 0))]
_layer_shapes = [jax.ShapeDtypeStruct((NP, H), jnp.float32),
                 jax.ShapeDtypeStruct((NP, H), jnp.float32)]


def _tc_layer_body(s0, s1, gp, dis, b, w, out, gn):
    d = dis[...]
    o = _postact(s0, s1, gp[...], d, b)
    out[...] = o
    gn[...] = d * jnp.dot(o, w[...], preferred_element_type=jnp.float32)


_tc_layer = pl.pallas_call(
    _tc_layer_body, grid=(GRID,),
    in_specs=_layer_specs, out_specs=_layer_outs, out_shape=_layer_shapes,
)


def _tc_layer1_body(s0, s1, hp, dis, b, w, out, gn):
    d = dis[...]
    o = _postact(s0, s1, d * hp[...], d, b)
    out[...] = o
    gn[...] = d * jnp.dot(o, w[...], preferred_element_type=jnp.float32)


_tc_layer1 = pl.pallas_call(
    _tc_layer1_body, grid=(GRID,),
    in_specs=_layer_specs, out_specs=_layer_outs, out_shape=_layer_shapes,
)


def _tc_final_body(s0, s1, g3, dis, b, o1, o2, wlin, blin, out):
    o3 = _postact(s0, s1, g3[...], dis[...], b)
    w = wlin[...]
    acc = jnp.dot(o1[...], w[0:H], preferred_element_type=jnp.float32)
    acc = acc + jnp.dot(o2[...], w[H:2 * H], preferred_element_type=jnp.float32)
    acc = acc + jnp.dot(o3, w[2 * H:3 * H], preferred_element_type=jnp.float32)
    out[...] = acc + blin[...]


_tc_final = pl.pallas_call(
    _tc_final_body,
    grid=(GRID,),
    in_specs=[pl.BlockSpec((R, H), lambda i: (i, 0)),
              pl.BlockSpec((R, H), lambda i: (i, 0)),
              pl.BlockSpec((R, H), lambda i: (i, 0)),
              pl.BlockSpec((R, 1), lambda i: (i, 0)),
              pl.BlockSpec((1, H), lambda i: (0, 0)),
              pl.BlockSpec((R, H), lambda i: (i, 0)),
              pl.BlockSpec((R, H), lambda i: (i, 0)),
              pl.BlockSpec((3 * H, C), lambda i: (0, 0)),
              pl.BlockSpec((1, C), lambda i: (0, 0))],
    out_specs=pl.BlockSpec((R, C), lambda i: (i, 0)),
    out_shape=jax.ShapeDtypeStruct((NP, C), jnp.float32),
)


def kernel(x, edge_index, edge_weights, W1, b1, W2, b2, W3, b3, Wlin, blin):
    src = edge_index[0]
    dst = edge_index[1]
    pad_e = EP - E
    pad_idx = (jnp.arange(pad_e, dtype=jnp.int32) * 997) % N
    src_p = jnp.concatenate([src, pad_idx])
    dst_p = jnp.concatenate([dst, pad_idx])
    ew_p = jnp.concatenate([edge_weights, jnp.zeros((pad_e,), jnp.float32)])
    x_p = jnp.pad(x, ((0, NP - N), (0, 0)))

    h1 = _tc_h1(x_p, W1)
    s1, disc = _sc_deg_scatter(h1, src_p, dst_p, ew_p)
    dis = disc[0].reshape(NP, 1)
    out1, g2 = _tc_layer1(s1[0], s1[1], h1, dis, b1.reshape(1, H), W2)
    s2 = _sc_scatter(g2, src_p, dst_p, ew_p)
    out2, g3 = _tc_layer(s2[0], s2[1], g2, dis, b2.reshape(1, H), W3)
    s3 = _sc_scatter(g3, src_p, dst_p, ew_p)
    logits = _tc_final(s3[0], s3[1], g3, dis, b3.reshape(1, H),
                       out1, out2, Wlin, blin.reshape(1, C))
    return logits[:N]
